# Initial kernel scaffold; baseline (speedup 1.0000x reference)
import jax
import jax.numpy as jnp
from jax.experimental import pallas as pl


def kernel(x_user, x_content, user_lin_w, user_lin_b, content_lin_w, content_lin_b, user_emb, content_emb, c1_uc_wl, c1_uc_bl, c1_uc_wr, c1_cu_wl, c1_cu_bl, c1_cu_wr, c2_uc_wl, c2_uc_bl, c2_uc_wr, c2_cu_wl, c2_cu_bl, c2_cu_wr, edge_index_uc, edge_index_cu, edge_label_index):
    raise NotImplementedError("write your pallas kernel here")


# speedup vs baseline: 2.0885x; 2.0885x over previous
import functools

import jax
import jax.numpy as jnp
from jax import lax
from jax.experimental import pallas as pl
from jax.experimental.pallas import tpu as pltpu
from jax.experimental.pallas import tpu_sc as plsc

NU = 10000
NC = 10000
E = 320000
EL = 100000
F = 128
H = 128

SC_CORES = 2
SC_TILES = 16

CHUNK = 128
E_PER_TILE = 20480
E_PAD = E_PER_TILE * SC_TILES
N_CHUNKS = E_PER_TILE // CHUNK
ROWS_PER_TILE = 632
N_PAD = ROWS_PER_TILE * SC_TILES
DUMMY_ROW = 10000

EL_PER_TILE = 3200
EL_PAD = EL_PER_TILE * SC_CORES * SC_TILES
N_CHUNKS_CLS = EL_PER_TILE // CHUNK

MB = 1000


def _dot_t(a, b):
    return lax.dot_general(a, b, (((1,), (1,)), ((), ())),
                           preferred_element_type=jnp.float32)


def _phase_a_body(x_ref, w1_ref, b1_ref, emb_ref, wl_ref, wr_ref, bl_ref,
                  y_ref, r_ref):
    t = _dot_t(x_ref[...], w1_ref[...]) + b1_ref[...] + emb_ref[...]
    y_ref[...] = _dot_t(t, wl_ref[...])
    r_ref[...] = _dot_t(t, wr_ref[...]) + bl_ref[...]


def _phase_a(x, w1, b1, emb, wl, wr, bl):
    grid = (NU // MB,)
    row = pl.BlockSpec((MB, F), lambda i: (i, 0))
    full = pl.BlockSpec((H, F), lambda i: (0, 0))
    vec = pl.BlockSpec((1, H), lambda i: (0, 0))
    return pl.pallas_call(
        _phase_a_body,
        grid=grid,
        in_specs=[row, full, vec, row, full, full, vec],
        out_specs=[row, row],
        out_shape=[jax.ShapeDtypeStruct((NU, H), jnp.float32)] * 2,
    )(x, w1, b1.reshape(1, H), emb, wl, wr, bl.reshape(1, H))


def _phase_c_body(s_ref, cnt_ref, r1_ref, wl_ref, wr_ref, bl_ref,
                  y2_ref, r2_ref):
    cnt = jnp.maximum(cnt_ref[...][:, 0:1], 1.0)
    h = jnp.maximum(s_ref[...] / cnt + r1_ref[...], 0.0)
    y2_ref[...] = _dot_t(h, wl_ref[...])
    r2_ref[...] = _dot_t(h, wr_ref[...]) + bl_ref[...]


def _phase_c(s, cnt, r1, wl, wr, bl):
    grid = (NU // MB,)
    row = pl.BlockSpec((MB, H), lambda i: (i, 0))
    crow = pl.BlockSpec((MB, H), lambda i: (i, 0))
    full = pl.BlockSpec((H, H), lambda i: (0, 0))
    vec = pl.BlockSpec((1, H), lambda i: (0, 0))
    return pl.pallas_call(
        _phase_c_body,
        grid=grid,
        in_specs=[row, crow, row, full, full, vec],
        out_specs=[row, row],
        out_shape=[jax.ShapeDtypeStruct((NU, H), jnp.float32)] * 2,
    )(s, cnt, r1, wl, wr, bl.reshape(1, H))


def _phase_e_body(s_ref, cnt_ref, r2_ref, o_ref):
    cnt = jnp.maximum(cnt_ref[...][:, 0:1], 1.0)
    o_ref[...] = s_ref[...] / cnt + r2_ref[...]


def _phase_e(s, cnt, r2):
    grid = (NU // MB,)
    row = pl.BlockSpec((MB, H), lambda i: (i, 0))
    crow = pl.BlockSpec((MB, H), lambda i: (i, 0))
    return pl.pallas_call(
        _phase_e_body,
        grid=grid,
        in_specs=[row, crow, row],
        out_specs=row,
        out_shape=jax.ShapeDtypeStruct((NU, H), jnp.float32),
    )(s, cnt, r2)


def _make_seg_kernel():
    mesh = plsc.VectorSubcoreMesh(core_axis_name="c", subcore_axis_name="s")

    @functools.partial(
        pl.kernel,
        out_type=[jax.ShapeDtypeStruct((N_PAD, H), jnp.float32)] * 2,
        mesh=mesh,
        scratch_types=[
            pltpu.VMEM((CHUNK,), jnp.int32),
            pltpu.VMEM((CHUNK,), jnp.int32),
            pltpu.VMEM((CHUNK, H), jnp.float32),
            pltpu.VMEM_SHARED((N_PAD, H), jnp.float32),
            pltpu.SemaphoreType.DMA,
        ],
    )
    def seg(y_a, y_b, src_a, dst_a, src_b, dst_b, zacc,
            out_a, out_b, idx_v, dst_v, rows_v, acc_sh, sem):
        core = lax.axis_index("c")
        tile = lax.axis_index("s")
        r0 = tile * ROWS_PER_TILE

        pltpu.sync_copy(zacc.at[pl.ds(r0, ROWS_PER_TILE)],
                        acc_sh.at[pl.ds(r0, ROWS_PER_TILE)])
        plsc.subcore_barrier()

        def loop(y_hbm, src_hbm, dst_hbm):
            def chunk(i, carry):
                off = tile * E_PER_TILE + i * CHUNK
                pltpu.sync_copy(src_hbm.at[pl.ds(off, CHUNK)], idx_v)
                pltpu.async_copy(y_hbm.at[idx_v], rows_v, sem).wait()
                pltpu.sync_copy(dst_hbm.at[pl.ds(off, CHUNK)], dst_v)
                pltpu.sync_copy(rows_v, acc_sh.at[dst_v], add=True)
                return carry
            lax.fori_loop(0, N_CHUNKS, chunk, 0)

        @pl.when(core == 0)
        def _():
            loop(y_a, src_a, dst_a)

        @pl.when(core == 1)
        def _():
            loop(y_b, src_b, dst_b)

        plsc.subcore_barrier()

        def out_copy(s_out):
            ro = 0
            while ro < ROWS_PER_TILE:
                rn = min(CHUNK, ROWS_PER_TILE - ro)
                pltpu.sync_copy(acc_sh.at[pl.ds(r0 + ro, rn)],
                                rows_v.at[pl.ds(0, rn)])
                pltpu.sync_copy(rows_v.at[pl.ds(0, rn)],
                                s_out.at[pl.ds(r0 + ro, rn)])
                ro += rn

        @pl.when(core == 0)
        def _():
            out_copy(out_a)

        @pl.when(core == 1)
        def _():
            out_copy(out_b)

    return seg


_seg_kernel = _make_seg_kernel()


def _make_cnt_kernel():
    mesh = plsc.VectorSubcoreMesh(core_axis_name="c", subcore_axis_name="s")

    @functools.partial(
        pl.kernel,
        out_type=[jax.ShapeDtypeStruct((N_PAD, H), jnp.float32)] * 2,
        mesh=mesh,
        scratch_types=[
            pltpu.VMEM((CHUNK,), jnp.int32),
            pltpu.VMEM((CHUNK, H), jnp.float32),
            pltpu.VMEM_SHARED((N_PAD, H), jnp.float32),
        ],
    )
    def cnt(dst_a, dst_b, zacc, ones_hbm, out_a, out_b,
            dst_v, rows_v, acc_sh):
        core = lax.axis_index("c")
        tile = lax.axis_index("s")
        r0 = tile * ROWS_PER_TILE

        pltpu.sync_copy(zacc.at[pl.ds(r0, ROWS_PER_TILE)],
                        acc_sh.at[pl.ds(r0, ROWS_PER_TILE)])
        pltpu.sync_copy(ones_hbm, rows_v)
        plsc.subcore_barrier()

        def loop(dst_hbm):
            def chunk(i, carry):
                off = tile * E_PER_TILE + i * CHUNK
                pltpu.sync_copy(dst_hbm.at[pl.ds(off, CHUNK)], dst_v)
                pltpu.sync_copy(rows_v, acc_sh.at[dst_v], add=True)
                return carry
            lax.fori_loop(0, N_CHUNKS, chunk, 0)

        @pl.when(core == 0)
        def _():
            loop(dst_a)

        @pl.when(core == 1)
        def _():
            loop(dst_b)

        plsc.subcore_barrier()

        def out_copy(c_out):
            ro = 0
            while ro < ROWS_PER_TILE:
                rn = min(CHUNK, ROWS_PER_TILE - ro)
                pltpu.sync_copy(acc_sh.at[pl.ds(r0 + ro, rn)],
                                rows_v.at[pl.ds(0, rn)])
                pltpu.sync_copy(rows_v.at[pl.ds(0, rn)],
                                c_out.at[pl.ds(r0 + ro, rn)])
                ro += rn

        @pl.when(core == 0)
        def _():
            out_copy(out_a)

        @pl.when(core == 1)
        def _():
            out_copy(out_b)

    return cnt


_cnt_kernel = _make_cnt_kernel()


def _cls_body(ou_hbm, oc_hbm, ia_hbm, ib_hbm, pred_out,
              ia_v, ib_v, ra_v, rb_v, out_v, sem_a, sem_b):
    core = lax.axis_index("c")
    tile = lax.axis_index("s")
    wid = core * SC_TILES + tile
    base = wid * EL_PER_TILE
    lane = lax.iota(jnp.int32, 16)

    def chunk(i, carry):
        off = base + i * CHUNK
        pltpu.sync_copy(ia_hbm.at[pl.ds(off, CHUNK)], ia_v)
        pltpu.sync_copy(ib_hbm.at[pl.ds(off, CHUNK)], ib_v)
        cp_a = pltpu.async_copy(ou_hbm.at[ia_v], ra_v, sem_a)
        cp_b = pltpu.async_copy(oc_hbm.at[ib_v], rb_v, sem_b)
        cp_a.wait()
        cp_b.wait()

        def group(g, carry2):
            rows = g * 16 + lane
            dv = jnp.zeros((16,), jnp.float32)
            for c in range(H):
                col = jnp.full((16,), c, jnp.int32)
                av = plsc.load_gather(ra_v, [rows, col])
                bv = plsc.load_gather(rb_v, [rows, col])
                dv = dv + av * bv
            out_v[pl.ds(i * CHUNK + g * 16, 16)] = dv
            return carry2

        lax.fori_loop(0, CHUNK // 16, group, 0)
        return carry

    lax.fori_loop(0, N_CHUNKS_CLS, chunk, 0)
    pltpu.sync_copy(out_v, pred_out.at[pl.ds(base, EL_PER_TILE)])


_cls_kernel = pl.kernel(
    _cls_body,
    out_type=jax.ShapeDtypeStruct((EL_PAD,), jnp.float32),
    mesh=plsc.VectorSubcoreMesh(core_axis_name="c", subcore_axis_name="s"),
    compiler_params=pltpu.CompilerParams(needs_layout_passes=False),
    scratch_types=[
        pltpu.VMEM((CHUNK,), jnp.int32),
        pltpu.VMEM((CHUNK,), jnp.int32),
        pltpu.VMEM((CHUNK, H), jnp.float32),
        pltpu.VMEM((CHUNK, H), jnp.float32),
        pltpu.VMEM((EL_PER_TILE,), jnp.float32),
        pltpu.SemaphoreType.DMA,
        pltpu.SemaphoreType.DMA,
    ],
)


def _pad_edges(idx, n, pad_val):
    pad = jnp.full((n - idx.shape[0],), pad_val, jnp.int32)
    return jnp.concatenate([idx.astype(jnp.int32), pad])


def kernel(x_user, x_content, user_lin_w, user_lin_b, content_lin_w,
           content_lin_b, user_emb, content_emb,
           c1_uc_wl, c1_uc_bl, c1_uc_wr, c1_cu_wl, c1_cu_bl, c1_cu_wr,
           c2_uc_wl, c2_uc_bl, c2_uc_wr, c2_cu_wl, c2_cu_bl, c2_cu_wr,
           edge_index_uc, edge_index_cu, edge_label_index):
    src_cu = _pad_edges(edge_index_cu[0], E_PAD, 0)
    dst_cu = _pad_edges(edge_index_cu[1], E_PAD, DUMMY_ROW)
    src_uc = _pad_edges(edge_index_uc[0], E_PAD, 0)
    dst_uc = _pad_edges(edge_index_uc[1], E_PAD, DUMMY_ROW)
    la = _pad_edges(edge_label_index[0], EL_PAD, 0)
    lb = _pad_edges(edge_label_index[1], EL_PAD, 0)

    zacc = jnp.zeros((N_PAD, H), jnp.float32)
    ones_hbm = jnp.ones((CHUNK, H), jnp.float32)

    cnt_u, cnt_c = _cnt_kernel(dst_cu, dst_uc, zacc, ones_hbm)

    yu1, ru1 = _phase_a(x_user, user_lin_w, user_lin_b, user_emb,
                        c1_uc_wl, c1_cu_wr, c1_cu_bl)
    yc1, rc1 = _phase_a(x_content, content_lin_w, content_lin_b, content_emb,
                        c1_cu_wl, c1_uc_wr, c1_uc_bl)

    su1, sc1 = _seg_kernel(yc1, yu1, src_cu, dst_cu, src_uc, dst_uc, zacc)

    yu2, ru2 = _phase_c(su1, cnt_u, ru1, c2_uc_wl, c2_cu_wr, c2_cu_bl)
    yc2, rc2 = _phase_c(sc1, cnt_c, rc1, c2_cu_wl, c2_uc_wr, c2_uc_bl)

    su2, sc2 = _seg_kernel(yc2, yu2, src_cu, dst_cu, src_uc, dst_uc, zacc)

    ou = _phase_e(su2, cnt_u, ru2)
    oc = _phase_e(sc2, cnt_c, rc2)

    pred = _cls_kernel(ou, oc, la, lb)
    return pred[:EL]


# speedup vs baseline: 2.3178x; 1.1098x over previous
import functools

import jax
import jax.numpy as jnp
from jax import lax
from jax.experimental import pallas as pl
from jax.experimental.pallas import tpu as pltpu
from jax.experimental.pallas import tpu_sc as plsc

NU = 10000
NC = 10000
E = 320000
EL = 100000
F = 128
H = 128

SC_CORES = 2
SC_TILES = 16

CHUNK = 128
E_PER_TILE = 20480
E_PAD = E_PER_TILE * SC_TILES
N_CHUNKS = E_PER_TILE // CHUNK
ROWS_PER_TILE = 632
N_PAD = ROWS_PER_TILE * SC_TILES
DUMMY_ROW = 10000

EL_PER_TILE = 3200
EL_PAD = EL_PER_TILE * SC_CORES * SC_TILES
N_CHUNKS_CLS = EL_PER_TILE // CHUNK

MB = 1000


def _dot_t(a, b):
    return lax.dot_general(a, b, (((1,), (1,)), ((), ())),
                           preferred_element_type=jnp.float32)


def _phase_a_body(x_ref, w1_ref, b1_ref, emb_ref, wl_ref, wr_ref, bl_ref,
                  y_ref, r_ref):
    t = _dot_t(x_ref[...], w1_ref[...]) + b1_ref[...] + emb_ref[...]
    y_ref[...] = _dot_t(t, wl_ref[...])
    r_ref[...] = _dot_t(t, wr_ref[...]) + bl_ref[...]


def _phase_a(x, w1, b1, emb, wl, wr, bl):
    grid = (NU // MB,)
    row = pl.BlockSpec((MB, F), lambda i: (i, 0))
    full = pl.BlockSpec((H, F), lambda i: (0, 0))
    vec = pl.BlockSpec((1, H), lambda i: (0, 0))
    return pl.pallas_call(
        _phase_a_body,
        grid=grid,
        in_specs=[row, full, vec, row, full, full, vec],
        out_specs=[row, row],
        out_shape=[jax.ShapeDtypeStruct((NU, H), jnp.float32)] * 2,
    )(x, w1, b1.reshape(1, H), emb, wl, wr, bl.reshape(1, H))


def _phase_c_body(s_ref, cnt_ref, r1_ref, wl_ref, wr_ref, bl_ref,
                  y2_ref, r2_ref):
    cnt = jnp.maximum(cnt_ref[...][:, 0:1], 1.0)
    h = jnp.maximum(s_ref[...] / cnt + r1_ref[...], 0.0)
    y2_ref[...] = _dot_t(h, wl_ref[...])
    r2_ref[...] = _dot_t(h, wr_ref[...]) + bl_ref[...]


def _phase_c(s, cnt, r1, wl, wr, bl):
    grid = (NU // MB,)
    row = pl.BlockSpec((MB, H), lambda i: (i, 0))
    crow = pl.BlockSpec((MB, H), lambda i: (i, 0))
    full = pl.BlockSpec((H, H), lambda i: (0, 0))
    vec = pl.BlockSpec((1, H), lambda i: (0, 0))
    return pl.pallas_call(
        _phase_c_body,
        grid=grid,
        in_specs=[row, crow, row, full, full, vec],
        out_specs=[row, row],
        out_shape=[jax.ShapeDtypeStruct((NU, H), jnp.float32)] * 2,
    )(s, cnt, r1, wl, wr, bl.reshape(1, H))


def _phase_e_body(s_ref, cnt_ref, r2_ref, o_ref):
    cnt = jnp.maximum(cnt_ref[...][:, 0:1], 1.0)
    o_ref[...] = s_ref[...] / cnt + r2_ref[...]


def _phase_e(s, cnt, r2):
    grid = (NU // MB,)
    row = pl.BlockSpec((MB, H), lambda i: (i, 0))
    crow = pl.BlockSpec((MB, H), lambda i: (i, 0))
    return pl.pallas_call(
        _phase_e_body,
        grid=grid,
        in_specs=[row, crow, row],
        out_specs=row,
        out_shape=jax.ShapeDtypeStruct((NU, H), jnp.float32),
    )(s, cnt, r2)


IDXB = 16
NBLK = N_CHUNKS // IDXB


def _make_seg_kernel():
    mesh = plsc.VectorSubcoreMesh(core_axis_name="c", subcore_axis_name="s")

    @functools.partial(
        pl.kernel,
        out_type=[jax.ShapeDtypeStruct((N_PAD, H), jnp.float32)] * 2,
        mesh=mesh,
        scratch_types=[
            pltpu.VMEM((IDXB, CHUNK), jnp.int32),
            pltpu.VMEM((IDXB, CHUNK), jnp.int32),
            pltpu.VMEM((CHUNK, H), jnp.float32),
            pltpu.VMEM((CHUNK, H), jnp.float32),
            pltpu.VMEM_SHARED((N_PAD, H), jnp.float32),
            pltpu.SemaphoreType.DMA,
            pltpu.SemaphoreType.DMA,
            pltpu.SemaphoreType.DMA,
            pltpu.SemaphoreType.DMA,
        ],
    )
    def seg(y_a, y_b, src_a, dst_a, src_b, dst_b, zacc,
            out_a, out_b, sidx_v, didx_v, rows0, rows1, acc_sh,
            gs0, gs1, ss0, ss1):
        core = lax.axis_index("c")
        tile = lax.axis_index("s")
        r0 = tile * ROWS_PER_TILE

        pltpu.sync_copy(zacc.at[pl.ds(r0, ROWS_PER_TILE)],
                        acc_sh.at[pl.ds(r0, ROWS_PER_TILE)])
        plsc.subcore_barrier()

        def loop(y_hbm, src2d, dst2d):
            for bi in range(NBLK):
                brow = tile * N_CHUNKS + bi * IDXB
                pltpu.sync_copy(src2d.at[pl.ds(brow, IDXB)], sidx_v)
                pltpu.sync_copy(dst2d.at[pl.ds(brow, IDXB)], didx_v)

                def pair(p, carry):
                    j0 = 2 * p
                    j1 = 2 * p + 1
                    g0 = pltpu.async_copy(y_hbm.at[sidx_v.at[j0]], rows0, gs0)
                    g1 = pltpu.async_copy(y_hbm.at[sidx_v.at[j1]], rows1, gs1)
                    g0.wait()
                    s0 = pltpu.async_copy(rows0, acc_sh.at[didx_v.at[j0]],
                                          ss0, add=True)
                    g1.wait()
                    s1 = pltpu.async_copy(rows1, acc_sh.at[didx_v.at[j1]],
                                          ss1, add=True)
                    s0.wait()
                    s1.wait()
                    return carry

                lax.fori_loop(0, IDXB // 2, pair, 0)

        @pl.when(core == 0)
        def _():
            loop(y_a, src_a, dst_a)

        @pl.when(core == 1)
        def _():
            loop(y_b, src_b, dst_b)

        plsc.subcore_barrier()

        def out_copy(s_out):
            ro = 0
            while ro < ROWS_PER_TILE:
                rn = min(CHUNK, ROWS_PER_TILE - ro)
                pltpu.sync_copy(acc_sh.at[pl.ds(r0 + ro, rn)],
                                rows0.at[pl.ds(0, rn)])
                pltpu.sync_copy(rows0.at[pl.ds(0, rn)],
                                s_out.at[pl.ds(r0 + ro, rn)])
                ro += rn

        @pl.when(core == 0)
        def _():
            out_copy(out_a)

        @pl.when(core == 1)
        def _():
            out_copy(out_b)

    return seg


_seg_kernel = _make_seg_kernel()


def _make_cnt_kernel():
    mesh = plsc.VectorSubcoreMesh(core_axis_name="c", subcore_axis_name="s")

    @functools.partial(
        pl.kernel,
        out_type=[jax.ShapeDtypeStruct((N_PAD, H), jnp.float32)] * 2,
        mesh=mesh,
        scratch_types=[
            pltpu.VMEM((IDXB, CHUNK), jnp.int32),
            pltpu.VMEM((CHUNK, H), jnp.float32),
            pltpu.VMEM_SHARED((N_PAD, H), jnp.float32),
            pltpu.SemaphoreType.DMA,
            pltpu.SemaphoreType.DMA,
        ],
    )
    def cnt(dst_a, dst_b, zacc, ones_hbm, out_a, out_b,
            didx_v, rows_v, acc_sh, ss0, ss1):
        core = lax.axis_index("c")
        tile = lax.axis_index("s")
        r0 = tile * ROWS_PER_TILE

        pltpu.sync_copy(zacc.at[pl.ds(r0, ROWS_PER_TILE)],
                        acc_sh.at[pl.ds(r0, ROWS_PER_TILE)])
        pltpu.sync_copy(ones_hbm, rows_v)
        plsc.subcore_barrier()

        def loop(dst2d):
            for bi in range(NBLK):
                brow = tile * N_CHUNKS + bi * IDXB
                pltpu.sync_copy(dst2d.at[pl.ds(brow, IDXB)], didx_v)

                def pair(p, carry):
                    s0 = pltpu.async_copy(rows_v, acc_sh.at[didx_v.at[2 * p]],
                                          ss0, add=True)
                    s1 = pltpu.async_copy(rows_v,
                                          acc_sh.at[didx_v.at[2 * p + 1]],
                                          ss1, add=True)
                    s0.wait()
                    s1.wait()
                    return carry

                lax.fori_loop(0, IDXB // 2, pair, 0)

        @pl.when(core == 0)
        def _():
            loop(dst_a)

        @pl.when(core == 1)
        def _():
            loop(dst_b)

        plsc.subcore_barrier()

        def out_copy(c_out):
            ro = 0
            while ro < ROWS_PER_TILE:
                rn = min(CHUNK, ROWS_PER_TILE - ro)
                pltpu.sync_copy(acc_sh.at[pl.ds(r0 + ro, rn)],
                                rows_v.at[pl.ds(0, rn)])
                pltpu.sync_copy(rows_v.at[pl.ds(0, rn)],
                                c_out.at[pl.ds(r0 + ro, rn)])
                ro += rn

        @pl.when(core == 0)
        def _():
            out_copy(out_a)

        @pl.when(core == 1)
        def _():
            out_copy(out_b)

    return cnt


_cnt_kernel = _make_cnt_kernel()


def _cls_body(ou_hbm, oc_hbm, ia_hbm, ib_hbm, pred_out,
              ia_v, ib_v, ra_v, rb_v, out_v, sem_a, sem_b):
    core = lax.axis_index("c")
    tile = lax.axis_index("s")
    wid = core * SC_TILES + tile
    base = wid * EL_PER_TILE
    lane = lax.iota(jnp.int32, 16)

    def chunk(i, carry):
        off = base + i * CHUNK
        pltpu.sync_copy(ia_hbm.at[pl.ds(off, CHUNK)], ia_v)
        pltpu.sync_copy(ib_hbm.at[pl.ds(off, CHUNK)], ib_v)
        cp_a = pltpu.async_copy(ou_hbm.at[ia_v], ra_v, sem_a)
        cp_b = pltpu.async_copy(oc_hbm.at[ib_v], rb_v, sem_b)
        cp_a.wait()
        cp_b.wait()

        def group(g, carry2):
            rows = g * 16 + lane
            dv = jnp.zeros((16,), jnp.float32)
            for c in range(H):
                col = jnp.full((16,), c, jnp.int32)
                av = plsc.load_gather(ra_v, [rows, col])
                bv = plsc.load_gather(rb_v, [rows, col])
                dv = dv + av * bv
            out_v[pl.ds(i * CHUNK + g * 16, 16)] = dv
            return carry2

        lax.fori_loop(0, CHUNK // 16, group, 0)
        return carry

    lax.fori_loop(0, N_CHUNKS_CLS, chunk, 0)
    pltpu.sync_copy(out_v, pred_out.at[pl.ds(base, EL_PER_TILE)])


_cls_kernel = pl.kernel(
    _cls_body,
    out_type=jax.ShapeDtypeStruct((EL_PAD,), jnp.float32),
    mesh=plsc.VectorSubcoreMesh(core_axis_name="c", subcore_axis_name="s"),
    compiler_params=pltpu.CompilerParams(needs_layout_passes=False),
    scratch_types=[
        pltpu.VMEM((CHUNK,), jnp.int32),
        pltpu.VMEM((CHUNK,), jnp.int32),
        pltpu.VMEM((CHUNK, H), jnp.float32),
        pltpu.VMEM((CHUNK, H), jnp.float32),
        pltpu.VMEM((EL_PER_TILE,), jnp.float32),
        pltpu.SemaphoreType.DMA,
        pltpu.SemaphoreType.DMA,
    ],
)


def _pad_edges(idx, n, pad_val):
    pad = jnp.full((n - idx.shape[0],), pad_val, jnp.int32)
    return jnp.concatenate([idx.astype(jnp.int32), pad])


def kernel(x_user, x_content, user_lin_w, user_lin_b, content_lin_w,
           content_lin_b, user_emb, content_emb,
           c1_uc_wl, c1_uc_bl, c1_uc_wr, c1_cu_wl, c1_cu_bl, c1_cu_wr,
           c2_uc_wl, c2_uc_bl, c2_uc_wr, c2_cu_wl, c2_cu_bl, c2_cu_wr,
           edge_index_uc, edge_index_cu, edge_label_index):
    n2d = E_PAD // CHUNK
    src_cu = _pad_edges(edge_index_cu[0], E_PAD, 0).reshape(n2d, CHUNK)
    dst_cu = _pad_edges(edge_index_cu[1], E_PAD, DUMMY_ROW).reshape(n2d, CHUNK)
    src_uc = _pad_edges(edge_index_uc[0], E_PAD, 0).reshape(n2d, CHUNK)
    dst_uc = _pad_edges(edge_index_uc[1], E_PAD, DUMMY_ROW).reshape(n2d, CHUNK)
    la = _pad_edges(edge_label_index[0], EL_PAD, 0)
    lb = _pad_edges(edge_label_index[1], EL_PAD, 0)

    zacc = jnp.zeros((N_PAD, H), jnp.float32)
    ones_hbm = jnp.ones((CHUNK, H), jnp.float32)

    cnt_u, cnt_c = _cnt_kernel(dst_cu, dst_uc, zacc, ones_hbm)

    yu1, ru1 = _phase_a(x_user, user_lin_w, user_lin_b, user_emb,
                        c1_uc_wl, c1_cu_wr, c1_cu_bl)
    yc1, rc1 = _phase_a(x_content, content_lin_w, content_lin_b, content_emb,
                        c1_cu_wl, c1_uc_wr, c1_uc_bl)

    su1, sc1 = _seg_kernel(yc1, yu1, src_cu, dst_cu, src_uc, dst_uc, zacc)

    yu2, ru2 = _phase_c(su1, cnt_u, ru1, c2_uc_wl, c2_cu_wr, c2_cu_bl)
    yc2, rc2 = _phase_c(sc1, cnt_c, rc1, c2_cu_wl, c2_uc_wr, c2_uc_bl)

    su2, sc2 = _seg_kernel(yc2, yu2, src_cu, dst_cu, src_uc, dst_uc, zacc)

    ou = _phase_e(su2, cnt_u, ru2)
    oc = _phase_e(sc2, cnt_c, rc2)

    pred = _cls_kernel(ou, oc, la, lb)
    return pred[:EL]
